# baseline (speedup 1.0000x reference)
import functools

import jax
import jax.numpy as jnp
from jax import lax
from jax.experimental import pallas as pl
from jax.experimental.pallas import tpu as pltpu
from jax.experimental.pallas import tpu_sc as plsc

N = 100000
E = 6400000
NC = 2
NS = 16
TILES = NC * NS
EPT = E // TILES
B = 2000
CH = EPT // B
NSLOT = 4
NP = 100096
RPT = NP // NS
CP = 8


def _make_sc_layer():
    mesh = plsc.VectorSubcoreMesh(core_axis_name="c", subcore_axis_name="s")

    @functools.partial(
        pl.kernel,
        mesh=mesh,
        compiler_params=pltpu.CompilerParams(use_tc_tiling_on_sc=False),
        out_type=jax.ShapeDtypeStruct((NC, NP, CP), jnp.float32),
        scratch_types=[
            pltpu.VMEM((NSLOT, B), jnp.int32),
            pltpu.VMEM((NSLOT, B), jnp.int32),
            pltpu.VMEM((NSLOT, B, CP), jnp.float32),
            pltpu.VMEM_SHARED((NP, CP), jnp.float32),
            pltpu.SemaphoreType.DMA,
            pltpu.SemaphoreType.DMA,
            pltpu.SemaphoreType.DMA,
        ],
    )
    def sc_layer(h_hbm, src_hbm, dst_hbm, zero_hbm, out_hbm,
                 src_v, dst_v, rows_v, acc_sh, sem_i, sem_g, sem_s):
        c = lax.axis_index("c")
        s = lax.axis_index("s")
        pltpu.sync_copy(zero_hbm.at[pl.ds(s * RPT, RPT)],
                        acc_sh.at[pl.ds(s * RPT, RPT)])
        plsc.subcore_barrier()

        tile_e0 = (c * NS + s) * EPT

        def issue_idx(k):
            sl = lax.rem(k, NSLOT)
            e0 = tile_e0 + k * B
            pltpu.async_copy(src_hbm.at[pl.ds(e0, B)], src_v.at[sl], sem_i)
            pltpu.async_copy(dst_hbm.at[pl.ds(e0, B)], dst_v.at[sl], sem_i)

        def wait_idx():
            pltpu.make_async_copy(src_hbm.at[pl.ds(0, B)], src_v.at[0], sem_i).wait()
            pltpu.make_async_copy(dst_hbm.at[pl.ds(0, B)], dst_v.at[0], sem_i).wait()

        def issue_gather(k):
            sl = lax.rem(k, NSLOT)
            pltpu.async_copy(h_hbm.at[src_v.at[sl]], rows_v.at[sl], sem_g)

        def wait_gather():
            pltpu.make_async_copy(h_hbm.at[src_v.at[0]], rows_v.at[0], sem_g).wait()

        def issue_scatter(k):
            sl = lax.rem(k, NSLOT)
            pltpu.async_copy(rows_v.at[sl], acc_sh.at[dst_v.at[sl]], sem_s,
                             add=True)

        def wait_scatter():
            pltpu.make_async_copy(rows_v.at[0], acc_sh.at[pl.ds(0, B)],
                                  sem_s).wait()

        issue_idx(0)
        issue_idx(1)

        def body(i, carry):
            @pl.when(i >= 2)
            def _():
                wait_scatter()

            @pl.when(i + 2 < CH)
            def _():
                issue_idx(i + 2)

            wait_idx()
            issue_gather(i)

            @pl.when(i >= 1)
            def _():
                wait_gather()
                issue_scatter(i - 1)

            return carry

        lax.fori_loop(0, CH, body, 0)
        wait_gather()
        issue_scatter(CH - 1)
        wait_scatter()
        wait_scatter()
        plsc.subcore_barrier()
        pltpu.sync_copy(acc_sh.at[pl.ds(s * RPT, RPT)],
                        out_hbm.at[c, pl.ds(s * RPT, RPT)])

    return sc_layer


_FG = 13
_FR = 968


def _bd(w, g):
    return jnp.kron(jnp.eye(g, dtype=jnp.float32), w)


def _pad_wt(w, rows, cols):
    wt = w.T
    return jnp.zeros((rows, cols), jnp.float32).at[:wt.shape[0], :wt.shape[1]].set(wt)


def _pad_b(b, cols):
    return jnp.zeros((1, cols), jnp.float32).at[0, :b.shape[0]].set(b)


def _tile16(b):
    return jnp.tile(b, (1, 16))


def _combine(a0, a1, h, bds, bdos, biases):
    nw = len(bds)

    def body(*refs):
        a0_ref, a1_ref, h_ref = refs[0], refs[1], refs[2]
        w_refs = refs[3:3 + nw]
        wo_refs = refs[3 + nw:3 + 2 * nw]
        b_refs = refs[3 + 2 * nw:3 + 3 * nw]
        o_refs = refs[3 + 3 * nw:]
        agg = a0_ref[...] + a1_ref[...]
        hh = h_ref[...]
        for w_ref, wo_ref, b_ref, o_ref in zip(w_refs, wo_refs, b_refs, o_refs):
            o_ref[...] = jnp.maximum(
                jnp.dot(agg, w_ref[...], preferred_element_type=jnp.float32)
                + jnp.dot(hh, wo_ref[...], preferred_element_type=jnp.float32)
                + b_ref[...], 0.0)

    out_shape = [jax.ShapeDtypeStruct((NP // 16, 128), jnp.float32)] * nw
    if nw == 1:
        out_shape = out_shape[0]
    return pl.pallas_call(body, out_shape=out_shape)(
        a0, a1, h, *bds, *bdos, *biases)


def _tc_final(a4a0, a4a1, a4b0, a4b1, h3a, h3b, bda, bdb, bdoa, bdob, b4t,
              bw1, bb1, bw2, bb2, bw3, bb3):
    def body(aa0, aa1, ab0, ab1, ha_ref, hb_ref, wa, wb, woa, wob, b_ref,
             w1_ref, b1_ref, w2_ref, b2_ref, w3_ref, b3_ref, o_ref):
        agg_a = aa0[...] + aa1[...]
        agg_b = ab0[...] + ab1[...]
        h4 = jnp.maximum(
            jnp.dot(agg_a, wa[...], preferred_element_type=jnp.float32)
            + jnp.dot(agg_b, wb[...], preferred_element_type=jnp.float32)
            + jnp.dot(ha_ref[...], woa[...], preferred_element_type=jnp.float32)
            + jnp.dot(hb_ref[...], wob[...], preferred_element_type=jnp.float32)
            + b_ref[...], 0.0)
        t = jnp.maximum(jnp.dot(h4, w1_ref[...],
                                preferred_element_type=jnp.float32) + b1_ref[...], 0.0)
        t = jnp.maximum(jnp.dot(t, w2_ref[...],
                                preferred_element_type=jnp.float32) + b2_ref[...], 0.0)
        o_ref[...] = jnp.dot(t, w3_ref[...],
                             preferred_element_type=jnp.float32) + b3_ref[...]

    full = lambda r, c: pl.BlockSpec((r, c), lambda i: (0, 0))
    return pl.pallas_call(
        body,
        grid=(_FG,),
        in_specs=[pl.BlockSpec((_FR, 64), lambda i: (i, 0))] * 6 + [
            full(64, 128), full(64, 128), full(64, 128), full(64, 128),
            full(1, 128),
            full(128, 256), full(1, 256),
            full(256, 128), full(1, 128),
            full(128, 1024), full(1, 1024),
        ],
        out_specs=pl.BlockSpec((_FR, 1024), lambda i: (i, 0)),
        out_shape=jax.ShapeDtypeStruct((N // 8, 1024), jnp.float32),
    )(a4a0, a4a1, a4b0, a4b1, h3a, h3b, bda, bdb, bdoa, bdob, b4t,
      bw1, bb1, bw2, bb2, bw3, bb3)


def kernel(x, edge_index, W_rel1, b_rel1, W_root1, W_rel2, b_rel2, W_root2,
           W_rel3, b_rel3, W_root3, W_rel4, b_rel4, W_root4,
           Wf1, bf1, Wf2, bf2, Wf3, bf3):
    src1d = edge_index[0]
    dst1d = edge_index[1]
    zeros8 = jnp.zeros((NP, CP), jnp.float32)

    sc8 = _make_sc_layer()

    pk = lambda a: a.reshape(NP // 16, 128)
    unpk = lambda p: p.reshape(NP, 8)[:N]

    x2 = jnp.pad(x[:, 0], (0, NP - N)).reshape(NP // 16, 16)
    onehot = jnp.zeros((8,), jnp.float32).at[0].set(1.0)
    h0p = (x2[:, :, None] * onehot).reshape(NP // 16, 128)

    acc = sc8(unpk(h0p), src1d, dst1d, zeros8)
    h1p = _combine(pk(acc[0]), pk(acc[1]), h0p,
                   [_bd(_pad_wt(W_rel1, 8, 8), 16)],
                   [_bd(_pad_wt(W_root1, 8, 8), 16)],
                   [_tile16(_pad_b(b_rel1, 8))])
    acc = sc8(unpk(h1p), src1d, dst1d, zeros8)
    h2p = _combine(pk(acc[0]), pk(acc[1]), h1p,
                   [_bd(_pad_wt(W_rel2, 8, 8), 16)],
                   [_bd(_pad_wt(W_root2, 8, 8), 16)],
                   [_tile16(_pad_b(b_rel2, 8))])
    acc = sc8(unpk(h2p), src1d, dst1d, zeros8)
    w3 = _pad_wt(W_rel3, 8, 16)
    wo3 = _pad_wt(W_root3, 8, 16)
    b3 = _pad_b(b_rel3, 16)
    h3ap, h3bp = _combine(pk(acc[0]), pk(acc[1]), h2p,
                          [_bd(w3[:, :8], 16), _bd(w3[:, 8:], 16)],
                          [_bd(wo3[:, :8], 16), _bd(wo3[:, 8:], 16)],
                          [_tile16(b3[:, :8]), _tile16(b3[:, 8:])])
    acca = sc8(unpk(h3ap), src1d, dst1d, zeros8)
    accb = sc8(unpk(h3bp), src1d, dst1d, zeros8)
    w4 = _pad_wt(W_rel4, 16, 16)
    wo4 = _pad_wt(W_root4, 16, 16)
    t8 = lambda b: jnp.tile(b, (1, 8))
    r8 = lambda a: a.reshape(NP // 8, 64)
    out = _tc_final(r8(acca[0]), r8(acca[1]), r8(accb[0]), r8(accb[1]),
                    r8(h3ap), r8(h3bp),
                    _bd(w4[:8], 8), _bd(w4[8:], 8),
                    _bd(wo4[:8], 8), _bd(wo4[8:], 8),
                    t8(_pad_b(b_rel4, 16)),
                    _bd(_pad_wt(Wf1, 16, 32), 8), t8(_pad_b(bf1, 32)),
                    _bd(_pad_wt(Wf2, 32, 16), 8), t8(_pad_b(bf2, 16)),
                    _bd(_pad_wt(Wf3, 16, 128), 8), t8(_pad_b(bf3, 128)))
    return out.reshape(N, 128)


# speedup vs baseline: 1.0886x; 1.0886x over previous
import functools

import jax
import jax.numpy as jnp
from jax import lax
from jax.experimental import pallas as pl
from jax.experimental.pallas import tpu as pltpu
from jax.experimental.pallas import tpu_sc as plsc

N = 100000
E = 6400000
NC = 2
NS = 16
TILES = NC * NS
EPT = E // TILES
B = 2000
CH = EPT // B
NSLOT = 4
NP = 100096
RPT = NP // NS
CP = 8


def _make_sc_layer():
    mesh = plsc.VectorSubcoreMesh(core_axis_name="c", subcore_axis_name="s")

    @functools.partial(
        pl.kernel,
        mesh=mesh,
        compiler_params=pltpu.CompilerParams(use_tc_tiling_on_sc=False),
        out_type=jax.ShapeDtypeStruct((NC, NP, CP), jnp.float32),
        scratch_types=[
            pltpu.VMEM((NSLOT, B), jnp.int32),
            pltpu.VMEM((NSLOT, B), jnp.int32),
            pltpu.VMEM((NSLOT, B, CP), jnp.float32),
            pltpu.VMEM_SHARED((NP, CP), jnp.float32),
            pltpu.SemaphoreType.DMA,
            pltpu.SemaphoreType.DMA,
            pltpu.SemaphoreType.DMA,
        ],
    )
    def sc_layer(h_hbm, src_hbm, dst_hbm, zero_hbm, out_hbm,
                 src_v, dst_v, rows_v, acc_sh, sem_i, sem_g, sem_s):
        c = lax.axis_index("c")
        s = lax.axis_index("s")
        pltpu.sync_copy(zero_hbm.at[pl.ds(s * RPT, RPT)],
                        acc_sh.at[pl.ds(s * RPT, RPT)])
        plsc.subcore_barrier()

        tile_e0 = (c * NS + s) * EPT

        def issue_idx(k):
            sl = lax.rem(k, NSLOT)
            e0 = tile_e0 + k * B
            pltpu.async_copy(src_hbm.at[pl.ds(e0, B)], src_v.at[sl], sem_i)
            pltpu.async_copy(dst_hbm.at[pl.ds(e0, B)], dst_v.at[sl], sem_i)

        def wait_idx():
            pltpu.make_async_copy(src_hbm.at[pl.ds(0, B)], src_v.at[0], sem_i).wait()
            pltpu.make_async_copy(dst_hbm.at[pl.ds(0, B)], dst_v.at[0], sem_i).wait()

        def issue_gather(k):
            sl = lax.rem(k, NSLOT)
            pltpu.async_copy(h_hbm.at[src_v.at[sl]], rows_v.at[sl], sem_g)

        def wait_gather():
            pltpu.make_async_copy(h_hbm.at[src_v.at[0]], rows_v.at[0], sem_g).wait()

        def issue_scatter(k):
            sl = lax.rem(k, NSLOT)
            pltpu.async_copy(rows_v.at[sl], acc_sh.at[dst_v.at[sl]], sem_s,
                             add=True)

        def wait_scatter():
            pltpu.make_async_copy(rows_v.at[0], acc_sh.at[pl.ds(0, B)],
                                  sem_s).wait()

        issue_idx(0)
        issue_idx(1)

        def body(i, carry):
            @pl.when(i >= 2)
            def _():
                wait_scatter()

            @pl.when(i + 2 < CH)
            def _():
                issue_idx(i + 2)

            wait_idx()
            issue_gather(i)

            @pl.when(i >= 1)
            def _():
                wait_gather()
                issue_scatter(i - 1)

            return carry

        lax.fori_loop(0, CH, body, 0)
        wait_gather()
        issue_scatter(CH - 1)
        wait_scatter()
        wait_scatter()
        plsc.subcore_barrier()
        pltpu.sync_copy(acc_sh.at[pl.ds(s * RPT, RPT)],
                        out_hbm.at[c, pl.ds(s * RPT, RPT)])

    return sc_layer


_R = 2000


def _relu_affine(a_ref, h_ref, wr_ref, wo_ref, b_ref):
    agg = a_ref[0] + a_ref[1]
    return jnp.maximum(
        jnp.dot(agg, wr_ref[...], preferred_element_type=jnp.float32)
        + jnp.dot(h_ref[...], wo_ref[...], preferred_element_type=jnp.float32)
        + b_ref[...], 0.0)


def _tc_combine(h, accs, wrel, wroot, b):
    cin = h.shape[1]
    cout = wrel.shape[1]

    def body(a_ref, h_ref, wr_ref, wo_ref, b_ref, o_ref):
        o_ref[...] = _relu_affine(a_ref, h_ref, wr_ref, wo_ref, b_ref)

    return pl.pallas_call(
        body,
        grid=(N // _R,),
        in_specs=[
            pl.BlockSpec((2, _R, cin), lambda i: (0, i, 0)),
            pl.BlockSpec((_R, cin), lambda i: (i, 0)),
            pl.BlockSpec((cin, cout), lambda i: (0, 0)),
            pl.BlockSpec((cin, cout), lambda i: (0, 0)),
            pl.BlockSpec((1, cout), lambda i: (0, 0)),
        ],
        out_specs=pl.BlockSpec((_R, cout), lambda i: (i, 0)),
        out_shape=jax.ShapeDtypeStruct((N, cout), jnp.float32),
    )(accs, h, wrel, wroot, b)


def _tc_combine_split(h, accs, wrel, wroot, b):
    cin = h.shape[1]

    def body(a_ref, h_ref, wr_ref, wo_ref, b_ref, oa_ref, ob_ref):
        res = _relu_affine(a_ref, h_ref, wr_ref, wo_ref, b_ref)
        oa_ref[...] = res[:, :8]
        ob_ref[...] = res[:, 8:]

    return pl.pallas_call(
        body,
        grid=(N // _R,),
        in_specs=[
            pl.BlockSpec((2, _R, cin), lambda i: (0, i, 0)),
            pl.BlockSpec((_R, cin), lambda i: (i, 0)),
            pl.BlockSpec((cin, 16), lambda i: (0, 0)),
            pl.BlockSpec((cin, 16), lambda i: (0, 0)),
            pl.BlockSpec((1, 16), lambda i: (0, 0)),
        ],
        out_specs=[pl.BlockSpec((_R, 8), lambda i: (i, 0)),
                   pl.BlockSpec((_R, 8), lambda i: (i, 0))],
        out_shape=[jax.ShapeDtypeStruct((N, 8), jnp.float32),
                   jax.ShapeDtypeStruct((N, 8), jnp.float32)],
    )(accs, h, wrel, wroot, b)


def _tc_final(ha, hb, acca, accb, wra, wrb, woa, wob, b,
              wf1, bf1, wf2, bf2, wf3, bf3):
    def body(aa_ref, ab_ref, ha_ref, hb_ref, wra_ref, wrb_ref,
             woa_ref, wob_ref, b_ref,
             w1_ref, b1_ref, w2_ref, b2_ref, w3_ref, b3_ref, o_ref):
        agg_a = aa_ref[0] + aa_ref[1]
        agg_b = ab_ref[0] + ab_ref[1]
        h4 = jnp.maximum(
            jnp.dot(agg_a, wra_ref[...], preferred_element_type=jnp.float32)
            + jnp.dot(agg_b, wrb_ref[...], preferred_element_type=jnp.float32)
            + jnp.dot(ha_ref[...], woa_ref[...], preferred_element_type=jnp.float32)
            + jnp.dot(hb_ref[...], wob_ref[...], preferred_element_type=jnp.float32)
            + b_ref[...], 0.0)
        t = jnp.maximum(jnp.dot(h4, w1_ref[...],
                                preferred_element_type=jnp.float32) + b1_ref[...], 0.0)
        t = jnp.maximum(jnp.dot(t, w2_ref[...],
                                preferred_element_type=jnp.float32) + b2_ref[...], 0.0)
        o_ref[...] = jnp.dot(t, w3_ref[...],
                             preferred_element_type=jnp.float32) + b3_ref[...]

    full = lambda r, c: pl.BlockSpec((r, c), lambda i: (0, 0))
    return pl.pallas_call(
        body,
        grid=(N // _R,),
        in_specs=[
            pl.BlockSpec((2, _R, 8), lambda i: (0, i, 0)),
            pl.BlockSpec((2, _R, 8), lambda i: (0, i, 0)),
            pl.BlockSpec((_R, 8), lambda i: (i, 0)),
            pl.BlockSpec((_R, 8), lambda i: (i, 0)),
            full(8, 16), full(8, 16), full(8, 16), full(8, 16), full(1, 16),
            full(16, 32), full(1, 32),
            full(32, 16), full(1, 16),
            full(16, 128), full(1, 128),
        ],
        out_specs=pl.BlockSpec((_R, 128), lambda i: (i, 0)),
        out_shape=jax.ShapeDtypeStruct((N, 128), jnp.float32),
    )(acca, accb, ha, hb, wra, wrb, woa, wob, b,
      wf1, bf1, wf2, bf2, wf3, bf3)


def _pad_w(w, rows, cols):
    wt = w.T
    return jnp.zeros((rows, cols), jnp.float32).at[:wt.shape[0], :wt.shape[1]].set(wt)


def _pad_b(b, cols):
    return jnp.zeros((1, cols), jnp.float32).at[0, :b.shape[0]].set(b)


def kernel(x, edge_index, W_rel1, b_rel1, W_root1, W_rel2, b_rel2, W_root2,
           W_rel3, b_rel3, W_root3, W_rel4, b_rel4, W_root4,
           Wf1, bf1, Wf2, bf2, Wf3, bf3):
    src1d = edge_index[0]
    dst1d = edge_index[1]
    zeros8 = jnp.zeros((NP, CP), jnp.float32)

    sc8 = _make_sc_layer()

    h0 = jnp.pad(x, ((0, 0), (0, 7)))
    acc1 = sc8(h0, src1d, dst1d, zeros8)
    h1 = _tc_combine(h0, acc1, _pad_w(W_rel1, 8, 8), _pad_w(W_root1, 8, 8),
                     _pad_b(b_rel1, 8))
    acc2 = sc8(h1, src1d, dst1d, zeros8)
    h2 = _tc_combine(h1, acc2, _pad_w(W_rel2, 8, 8), _pad_w(W_root2, 8, 8),
                     _pad_b(b_rel2, 8))
    acc3 = sc8(h2, src1d, dst1d, zeros8)
    h3a, h3b = _tc_combine_split(h2, acc3, _pad_w(W_rel3, 8, 16),
                                 _pad_w(W_root3, 8, 16),
                                 _pad_b(b_rel3, 16))
    acc4a = sc8(h3a, src1d, dst1d, zeros8)
    acc4b = sc8(h3b, src1d, dst1d, zeros8)
    w4 = _pad_w(W_rel4, 16, 16)
    wo4 = _pad_w(W_root4, 16, 16)
    out = _tc_final(h3a, h3b, acc4a, acc4b,
                    w4[:8], w4[8:], wo4[:8], wo4[8:], _pad_b(b_rel4, 16),
                    _pad_w(Wf1, 16, 32), _pad_b(bf1, 32),
                    _pad_w(Wf2, 32, 16), _pad_b(bf2, 16),
                    _pad_w(Wf3, 16, 128), _pad_b(bf3, 128))
    return out
